# baseline (speedup 1.0000x reference)
import dataclasses
import functools

import jax
import jax.numpy as jnp
from jax import lax
from jax.experimental import pallas as pl
from jax.experimental.pallas import tpu as pltpu
from jax.experimental.pallas import tpu_sc as plsc

B = 16384
D = 64
NROWS = 1000000
NGRP = NROWS // 8
NC = 2
NS = 16
NW = NC * NS
BPW = B // NW
L = 16
SHB = 15
CR = 384
NR = 8
STG = B + NW


def _popcnt(m):
    pc = plsc.all_reduce_population_count(m)
    return pc if pc.ndim == 0 else pc[0]


def _scatter_body(users_hbm, items_hbm, ut_hbm, it_hbm, ustg, istg,
                  vals, ndl_u, ndl_b, todo_u, todo_b, chunk, hit, hitb,
                  cnts, sem, *wsems):
    wid = lax.axis_index("s") * NC + lax.axis_index("c")
    lane_iota = lax.broadcasted_iota(jnp.int32, (L,), 0)
    lane0 = lane_iota == 0
    dummy = B + wid

    def flush(slot, idxv, stg_hbm):
        for s in range(NR):
            @pl.when(slot == s)
            def _():
                pltpu.async_copy(hit.at[s], stg_hbm.at[idxv], wsems[s])
        cnts[2] = cnts[2] + 1

    def drain_one(slot, stg_hbm):
        for s in range(NR):
            @pl.when(slot == s)
            def _():
                pltpu.make_async_copy(hit.at[s], stg_hbm.at[lane_iota + B],
                                      wsems[s]).wait()

    def scan_table(idx_hbm, tbl_hbm, stg_hbm):
        pltpu.sync_copy(idx_hbm, vals)
        cnts[0] = 0
        cnts[2] = 0

        @pl.loop(0, B, step=L)
        def _(i):
            u = vals[pl.ds(i, L)]
            m = (u >> SHB) == wid
            w = cnts[0]
            plsc.store_compressed(ndl_u.at[pl.ds(w, L)], u, mask=m)
            plsc.store_compressed(ndl_b.at[pl.ds(w, L)], lane_iota + i, mask=m)
            cnts[0] = w + _popcnt(m)

        ncnt = cnts[0]
        ntrip = (ncnt + L - 1) // L
        cnts[1] = 0

        def process_chunk(r0, row_lo, row_hi):
            r0 = pl.multiple_of(r0, 8)
            pltpu.sync_copy(tbl_hbm.at[pl.ds(r0, CR)], chunk)

            @pl.loop(0, ntrip)
            def _(j):
                u = ndl_u[pl.ds(j * L, L)]
                b = ndl_b[pl.ds(j * L, L)]
                valid = (j * L + lane_iota) < ncnt
                m = valid & (u >= row_lo) & (u < row_hi)
                pc = _popcnt(m)

                @pl.when(pc > 0)
                def _():
                    plsc.store_compressed(todo_u.at[pl.ds(0, L)], u, mask=m)
                    plsc.store_compressed(todo_b.at[pl.ds(0, L)], b, mask=m)
                    tu = todo_u[pl.ds(0, L)]
                    tb = todo_b[pl.ds(0, L)]
                    for k in range(L):
                        @pl.when(k < pc)
                        def _():
                            lr = tu[k] - r0
                            w = cnts[1]
                            slot = (w >> 4) & (NR - 1)
                            lane = w & (L - 1)

                            @pl.when((lane == 0) & (w >= L * NR))
                            def _():
                                drain_one(slot, stg_hbm)

                            for c in range(D // L):
                                hit[slot, lane, pl.ds(c * L, L)] = (
                                    chunk[lr, pl.ds(c * L, L)])
                            bs = plsc.bitcast(
                                jnp.full((L,), tb[k], jnp.int32), jnp.float32)
                            plsc.store_compressed(
                                hitb.at[slot, pl.ds(lane, L)], bs, mask=lane0)
                            cnts[1] = w + 1

                            @pl.when(lane == L - 1)
                            def _():
                                idxv = plsc.bitcast(
                                    hitb[slot, pl.ds(0, L)], jnp.int32)
                                flush(slot, idxv, stg_hbm)

        start_r = wid << SHB
        nrows = jnp.maximum(jnp.minimum(1 << SHB, NROWS - start_r), 0)
        nfull = nrows // CR
        rem = nrows - nfull * CR

        @pl.loop(0, nfull)
        def _(c):
            r0 = start_r + c * CR
            process_chunk(r0, r0, r0 + CR)

        @pl.when(rem > 0)
        def _():
            r0 = start_r + nrows - CR
            process_chunk(r0, start_r + nfull * CR, start_r + nrows)

        w = cnts[1]
        wrem = w & (L - 1)

        @pl.when(wrem > 0)
        def _():
            slot = (w >> 4) & (NR - 1)
            cur = plsc.bitcast(hitb[slot, pl.ds(0, L)], jnp.int32)
            idxv = jnp.where(lane_iota < wrem, cur, dummy)
            flush(slot, idxv, stg_hbm)

        nT = cnts[2]
        for q in range(NR):
            @pl.when(nT > q)
            def _():
                pltpu.make_async_copy(hit.at[q], stg_hbm.at[lane_iota + B],
                                      wsems[q]).wait()

    scan_table(users_hbm, ut_hbm, ustg)
    scan_table(items_hbm, it_hbm, istg)


def _compute_body(ustg, istg, out_hbm, ubuf, ibuf, outv, sem):
    wid = lax.axis_index("s") * NC + lax.axis_index("c")
    base = wid * BPW
    lane_iota = lax.broadcasted_iota(jnp.int32, (L,), 0)
    CC = 128

    for ch in range(BPW // CC):
        cu = pltpu.async_copy(ustg.at[pl.ds(base + ch * CC, CC)], ubuf, sem)
        ci = pltpu.async_copy(istg.at[pl.ds(base + ch * CC, CC)], ibuf, sem)
        cu.wait()
        ci.wait()

        @pl.loop(0, CC, step=L)
        def _(g):
            resv = jnp.zeros((L,), jnp.float32)
            for k in range(L):
                r = g + k
                acc = ubuf[r, pl.ds(0, L)] * ibuf[r, pl.ds(0, L)]
                for c in range(1, D // L):
                    acc = acc + (ubuf[r, pl.ds(c * L, L)]
                                 * ibuf[r, pl.ds(c * L, L)])
                resv = jnp.where(lane_iota == k, jnp.sum(acc), resv)
            outv[pl.ds(ch * CC + g, L)] = 1.0 / (1.0 + jnp.exp(-resv))

    pltpu.sync_copy(outv, out_hbm.at[pl.ds(base, BPW)])


@jax.jit
def kernel(users, items, user_table, item_table):
    mesh = plsc.VectorSubcoreMesh(core_axis_name="c", subcore_axis_name="s")
    cp = pltpu.CompilerParams()
    if "needs_layout_passes" in pltpu.CompilerParams.__dataclass_fields__:
        cp = dataclasses.replace(cp, needs_layout_passes=False)

    scatter_k = pl.kernel(
        _scatter_body,
        out_type=(jax.ShapeDtypeStruct((STG, 128), jnp.float32),
                  jax.ShapeDtypeStruct((STG, 128), jnp.float32)),
        mesh=mesh,
        compiler_params=cp,
        scratch_types=[
            pltpu.VMEM((B,), jnp.int32),
            pltpu.VMEM((B + L,), jnp.int32),
            pltpu.VMEM((B + L,), jnp.int32),
            pltpu.VMEM((2 * L,), jnp.int32),
            pltpu.VMEM((2 * L,), jnp.int32),
            pltpu.VMEM((CR, D), jnp.float32),
            pltpu.VMEM((NR, L, 128), jnp.float32),
            pltpu.VMEM((NR, 2 * L), jnp.float32),
            pltpu.SMEM((4,), jnp.int32),
            pltpu.SemaphoreType.DMA,
        ] + [pltpu.SemaphoreType.DMA] * NR,
    )
    compute_k = pl.kernel(
        _compute_body,
        out_type=jax.ShapeDtypeStruct((B,), jnp.float32),
        mesh=mesh,
        compiler_params=cp,
        scratch_types=[
            pltpu.VMEM((128, 128), jnp.float32),
            pltpu.VMEM((128, 128), jnp.float32),
            pltpu.VMEM((BPW,), jnp.float32),
            pltpu.SemaphoreType.DMA,
        ],
    )

    ustg, istg = scatter_k(users.astype(jnp.int32), items.astype(jnp.int32),
                           user_table, item_table)
    return compute_k(ustg, istg)


# speedup vs baseline: 1.2203x; 1.2203x over previous
import dataclasses
import functools

import jax
import jax.numpy as jnp
from jax import lax
from jax.experimental import pallas as pl
from jax.experimental.pallas import tpu as pltpu
from jax.experimental.pallas import tpu_sc as plsc
from jax.experimental import layout as jlayout

B = 16384
D = 64
NROWS = 1000000
NGRP = NROWS // 8
NC = 2
NS = 16
NW = NC * NS
BPW = B // NW
L = 16
SHB = 15
CG = 48
CR = CG * 8
NR = 8
STG = B + NW


def _popcnt(m):
    pc = plsc.all_reduce_population_count(m)
    return pc if pc.ndim == 0 else pc[0]


def _scatter_body(users_hbm, items_hbm, ut_hbm, it_hbm, ustg, istg,
                  vals, ndl_u, ndl_b, todo_u, todo_b, chunk, hit, hitb,
                  cnts, sem, *wsems):
    wid = lax.axis_index("s") * NC + lax.axis_index("c")
    lane_iota = lax.broadcasted_iota(jnp.int32, (L,), 0)
    lane0 = lane_iota == 0
    dummy = B + wid

    def flush(slot, idxv, stg_hbm):
        for s in range(NR):
            @pl.when(slot == s)
            def _():
                pltpu.async_copy(hit.at[s], stg_hbm.at[idxv], wsems[s])
        cnts[2] = cnts[2] + 1

    def drain_one(slot, stg_hbm):
        for s in range(NR):
            @pl.when(slot == s)
            def _():
                pltpu.make_async_copy(hit.at[s], stg_hbm.at[lane_iota + B],
                                      wsems[s]).wait()

    def scan_table(idx_hbm, tbl_hbm, stg_hbm):
        pltpu.sync_copy(idx_hbm, vals)
        cnts[0] = 0
        cnts[2] = 0

        @pl.loop(0, B, step=L)
        def _(i):
            u = vals[pl.ds(i, L)]
            m = (u >> SHB) == wid
            w = cnts[0]
            plsc.store_compressed(ndl_u.at[pl.ds(w, L)], u, mask=m)
            plsc.store_compressed(ndl_b.at[pl.ds(w, L)], lane_iota + i, mask=m)
            cnts[0] = w + _popcnt(m)

        ncnt = cnts[0]
        ntrip = (ncnt + L - 1) // L
        cnts[1] = 0

        def process_chunk(g0, row_lo, row_hi):
            r0 = g0 * 8
            pltpu.sync_copy(tbl_hbm.at[pl.ds(g0, CG)], chunk)

            @pl.loop(0, ntrip)
            def _(j):
                u = ndl_u[pl.ds(j * L, L)]
                b = ndl_b[pl.ds(j * L, L)]
                valid = (j * L + lane_iota) < ncnt
                m = valid & (u >= row_lo) & (u < row_hi)
                pc = _popcnt(m)

                @pl.when(pc > 0)
                def _():
                    plsc.store_compressed(todo_u.at[pl.ds(0, L)], u, mask=m)
                    plsc.store_compressed(todo_b.at[pl.ds(0, L)], b, mask=m)
                    tu = todo_u[pl.ds(0, L)]
                    tb = todo_b[pl.ds(0, L)]
                    for k in range(L):
                        @pl.when(k < pc)
                        def _():
                            lr = tu[k] - r0
                            w = cnts[1]
                            slot = (w >> 4) & (NR - 1)
                            lane = w & (L - 1)

                            @pl.when((lane == 0) & (w >= L * NR))
                            def _():
                                drain_one(slot, stg_hbm)

                            for c in range(D // L):
                                hit[slot, lane, pl.ds(c * L, L)] = (
                                    chunk[lr >> 3, lr & 7, pl.ds(c * L, L)])
                            bs = plsc.bitcast(
                                jnp.full((L,), tb[k], jnp.int32), jnp.float32)
                            plsc.store_compressed(
                                hitb.at[slot, pl.ds(lane, L)], bs, mask=lane0)
                            cnts[1] = w + 1

                            @pl.when(lane == L - 1)
                            def _():
                                idxv = plsc.bitcast(
                                    hitb[slot, pl.ds(0, L)], jnp.int32)
                                flush(slot, idxv, stg_hbm)

        start_g = wid << (SHB - 3)
        ng = jnp.maximum(jnp.minimum(1 << (SHB - 3), NGRP - start_g), 0)
        nfull = ng // CG
        rem = ng - nfull * CG

        @pl.loop(0, nfull)
        def _(c):
            g0 = start_g + c * CG
            process_chunk(g0, (g0) * 8, (g0 + CG) * 8)

        @pl.when(rem > 0)
        def _():
            g0 = start_g + ng - CG
            process_chunk(g0, (start_g + nfull * CG) * 8, (start_g + ng) * 8)

        w = cnts[1]
        wrem = w & (L - 1)

        @pl.when(wrem > 0)
        def _():
            slot = (w >> 4) & (NR - 1)
            cur = plsc.bitcast(hitb[slot, pl.ds(0, L)], jnp.int32)
            idxv = jnp.where(lane_iota < wrem, cur, dummy)
            flush(slot, idxv, stg_hbm)

        nT = cnts[2]
        for q in range(NR):
            @pl.when(nT > q)
            def _():
                pltpu.make_async_copy(hit.at[q], stg_hbm.at[lane_iota + B],
                                      wsems[q]).wait()

    scan_table(users_hbm, ut_hbm, ustg)
    scan_table(items_hbm, it_hbm, istg)


def _compute_body(ustg, istg, out_hbm, ubuf, ibuf, outv, sem):
    wid = lax.axis_index("s") * NC + lax.axis_index("c")
    base = wid * BPW
    lane_iota = lax.broadcasted_iota(jnp.int32, (L,), 0)
    CC = 128

    for ch in range(BPW // CC):
        cu = pltpu.async_copy(ustg.at[pl.ds(base + ch * CC, CC)], ubuf, sem)
        ci = pltpu.async_copy(istg.at[pl.ds(base + ch * CC, CC)], ibuf, sem)
        cu.wait()
        ci.wait()

        @pl.loop(0, CC, step=L)
        def _(g):
            resv = jnp.zeros((L,), jnp.float32)
            for k in range(L):
                r = g + k
                acc = ubuf[r, pl.ds(0, L)] * ibuf[r, pl.ds(0, L)]
                for c in range(1, D // L):
                    acc = acc + (ubuf[r, pl.ds(c * L, L)]
                                 * ibuf[r, pl.ds(c * L, L)])
                resv = jnp.where(lane_iota == k, jnp.sum(acc), resv)
            outv[pl.ds(ch * CC + g, L)] = 1.0 / (1.0 + jnp.exp(-resv))

    pltpu.sync_copy(outv, out_hbm.at[pl.ds(base, BPW)])


@jax.jit
def kernel(users, items, user_table, item_table):
    mesh = plsc.VectorSubcoreMesh(core_axis_name="c", subcore_axis_name="s")
    cp = pltpu.CompilerParams()
    if "needs_layout_passes" in pltpu.CompilerParams.__dataclass_fields__:
        cp = dataclasses.replace(cp, needs_layout_passes=False)

    scatter_k = pl.kernel(
        _scatter_body,
        out_type=(jax.ShapeDtypeStruct((STG, 128), jnp.float32),
                  jax.ShapeDtypeStruct((STG, 128), jnp.float32)),
        mesh=mesh,
        compiler_params=cp,
        scratch_types=[
            pltpu.VMEM((B,), jnp.int32),
            pltpu.VMEM((B + L,), jnp.int32),
            pltpu.VMEM((B + L,), jnp.int32),
            pltpu.VMEM((2 * L,), jnp.int32),
            pltpu.VMEM((2 * L,), jnp.int32),
            pltpu.VMEM((CG, 8, D), jnp.float32),
            pltpu.VMEM((NR, L, 128), jnp.float32),
            pltpu.VMEM((NR, 2 * L), jnp.float32),
            pltpu.SMEM((4,), jnp.int32),
            pltpu.SemaphoreType.DMA,
        ] + [pltpu.SemaphoreType.DMA] * NR,
    )
    compute_k = pl.kernel(
        _compute_body,
        out_type=jax.ShapeDtypeStruct((B,), jnp.float32),
        mesh=mesh,
        compiler_params=cp,
        scratch_types=[
            pltpu.VMEM((128, 128), jnp.float32),
            pltpu.VMEM((128, 128), jnp.float32),
            pltpu.VMEM((BPW,), jnp.float32),
            pltpu.SemaphoreType.DMA,
        ],
    )

    ut3 = user_table.reshape(NGRP, 8, D)
    it3 = item_table.reshape(NGRP, 8, D)
    ustg, istg = scatter_k(users.astype(jnp.int32), items.astype(jnp.int32),
                           ut3, it3)
    return compute_k(ustg, istg)


# speedup vs baseline: 1.9397x; 1.5896x over previous
import dataclasses
import functools

import jax
import jax.numpy as jnp
from jax import lax
from jax.experimental import pallas as pl
from jax.experimental.pallas import tpu as pltpu
from jax.experimental.pallas import tpu_sc as plsc

B = 16384
D = 64
NC = 2
NS = 16
NW = NC * NS
BPW = B // NW
CH = 256
L = 16


def _rec_mf_body(users_hbm, items_hbm, ut_hbm, it_hbm, out_hbm,
                 uidx, iidx, urows, irows, outv, sem):
    wid = lax.axis_index("s") * NC + lax.axis_index("c")
    base = wid * BPW

    pltpu.sync_copy(users_hbm.at[pl.ds(base, BPW)], uidx)
    pltpu.sync_copy(items_hbm.at[pl.ds(base, BPW)], iidx)

    lane_iota = lax.broadcasted_iota(jnp.int32, (L,), 0)

    for ch in range(BPW // CH):
        off = ch * CH

        @pl.loop(0, CH, step=L)
        def _(g):
            uvec = uidx[pl.ds(off + g, L)]
            ivec = iidx[pl.ds(off + g, L)]
            for k in range(L):
                pltpu.async_copy(ut_hbm.at[pl.ds(uvec[k], 1)],
                                 urows.at[pl.ds(g + k, 1)], sem)
                pltpu.async_copy(it_hbm.at[pl.ds(ivec[k], 1)],
                                 irows.at[pl.ds(g + k, 1)], sem)

        pltpu.make_async_copy(ut_hbm.at[pl.ds(0, CH)], urows, sem).wait()
        pltpu.make_async_copy(it_hbm.at[pl.ds(0, CH)], irows, sem).wait()

        @pl.loop(0, CH, step=L)
        def _(g):
            resv = jnp.zeros((L,), jnp.float32)
            for k in range(L):
                r = g + k
                acc = urows[r, pl.ds(0, L)] * irows[r, pl.ds(0, L)]
                for c in range(1, D // L):
                    acc = acc + (urows[r, pl.ds(c * L, L)]
                                 * irows[r, pl.ds(c * L, L)])
                resv = jnp.where(lane_iota == k, jnp.sum(acc), resv)
            outv[pl.ds(off + g, L)] = 1.0 / (1.0 + jnp.exp(-resv))

    pltpu.sync_copy(outv, out_hbm.at[pl.ds(base, BPW)])


@jax.jit
def kernel(users, items, user_table, item_table):
    mesh = plsc.VectorSubcoreMesh(core_axis_name="c", subcore_axis_name="s")
    cp = pltpu.CompilerParams()
    if "needs_layout_passes" in pltpu.CompilerParams.__dataclass_fields__:
        cp = dataclasses.replace(cp, needs_layout_passes=False)
    k = pl.kernel(
        _rec_mf_body,
        out_type=jax.ShapeDtypeStruct((B,), jnp.float32),
        mesh=mesh,
        compiler_params=cp,
        scratch_types=[
            pltpu.VMEM((BPW,), jnp.int32),
            pltpu.VMEM((BPW,), jnp.int32),
            pltpu.VMEM((CH, D), jnp.float32),
            pltpu.VMEM((CH, D), jnp.float32),
            pltpu.VMEM((BPW,), jnp.float32),
            pltpu.SemaphoreType.DMA,
        ],
    )
    return k(users.astype(jnp.int32), items.astype(jnp.int32),
             user_table, item_table)
